# Initial kernel scaffold; baseline (speedup 1.0000x reference)
import jax
import jax.numpy as jnp
from jax.experimental import pallas as pl


def kernel(inputs, emb_table, W, b):
    raise NotImplementedError("write your pallas kernel here")


# speedup vs baseline: 5.9649x; 5.9649x over previous
import functools

import jax
import jax.numpy as jnp
from jax import lax
from jax.experimental import pallas as pl
from jax.experimental.pallas import tpu as pltpu
from jax.experimental.pallas import tpu_sc as plsc

BATCH = 4096
SEQ = 200
EMBED_DIM = 128
LANES = 16
NREG = EMBED_DIM // LANES

NC = 2
NS = 16
NW = NC * NS
B_PER_W = BATCH // NW

CHUNK = 100
NCH = SEQ // CHUNK
CPW = B_PER_W * NCH


def _sc_pool_body(idx_hbm, table_hbm, out_hbm, idx_v, rows_v, stage_v, sem):
    wid = lax.axis_index("s") * NC + lax.axis_index("c")
    row0 = wid * B_PER_W
    pltpu.sync_copy(idx_hbm.at[pl.ds(wid * CPW, CPW)], idx_v)

    def row_body(r, carry):
        accs = tuple(jnp.zeros((LANES,), jnp.float32) for _ in range(NREG))
        for half in range(NCH):
            j = r * NCH + half
            pltpu.async_copy(table_hbm.at[idx_v.at[j]], rows_v, sem).wait()

            def acc_body(k, a):
                return tuple(
                    a[g] + rows_v[k, pl.ds(g * LANES, LANES)]
                    for g in range(NREG)
                )

            accs = lax.fori_loop(0, CHUNK, acc_body, accs)
        for g in range(NREG):
            stage_v[r, pl.ds(g * LANES, LANES)] = accs[g]
        return carry

    lax.fori_loop(0, B_PER_W, row_body, 0)
    pltpu.sync_copy(stage_v, out_hbm.at[pl.ds(row0, B_PER_W)])


@functools.lru_cache(maxsize=1)
def _make_sc_pool():
    return pl.kernel(
        _sc_pool_body,
        out_type=jax.ShapeDtypeStruct((BATCH, EMBED_DIM), jnp.float32),
        mesh=plsc.VectorSubcoreMesh(core_axis_name="c", subcore_axis_name="s"),
        scratch_types=[
            pltpu.VMEM((CPW, CHUNK), jnp.int32),
            pltpu.VMEM((CHUNK, EMBED_DIM), jnp.float32),
            pltpu.VMEM((B_PER_W, EMBED_DIM), jnp.float32),
            pltpu.SemaphoreType.DMA,
        ],
    )


def _tc_head_body(x_ref, w_ref, b_ref, o_ref):
    x = x_ref[...]
    w = w_ref[...]
    logits = lax.dot_general(
        x, w, (((1,), (1,)), ((), ())), preferred_element_type=jnp.float32
    )
    logits = logits + b_ref[...]
    m = jnp.max(logits, axis=1, keepdims=True)
    s = logits - m
    lse = jnp.log(jnp.sum(jnp.exp(s), axis=1, keepdims=True))
    o_ref[...] = s - lse


def _tc_head(pooled, W, b2d):
    n_classes = W.shape[0]
    bm = 512
    grid = (BATCH // bm,)
    return pl.pallas_call(
        _tc_head_body,
        grid=grid,
        in_specs=[
            pl.BlockSpec((bm, EMBED_DIM), lambda i: (i, 0)),
            pl.BlockSpec((n_classes, EMBED_DIM), lambda i: (0, 0)),
            pl.BlockSpec((1, n_classes), lambda i: (0, 0)),
        ],
        out_specs=pl.BlockSpec((bm, n_classes), lambda i: (i, 0)),
        out_shape=jax.ShapeDtypeStruct((BATCH, n_classes), jnp.float32),
    )(pooled, W, b2d)


def kernel(inputs, emb_table, W, b):
    idx2d = inputs.reshape(BATCH * SEQ // CHUNK, CHUNK).astype(jnp.int32)
    pooled = _make_sc_pool()(idx2d, emb_table)
    return _tc_head(pooled, W, b.reshape(1, -1))


# speedup vs baseline: 11.7381x; 1.9678x over previous
import functools

import jax
import jax.numpy as jnp
from jax import lax
from jax.experimental import pallas as pl
from jax.experimental.pallas import tpu as pltpu
from jax.experimental.pallas import tpu_sc as plsc

BATCH = 4096
SEQ = 200
EMBED_DIM = 128
LANES = 16
NREG = EMBED_DIM // LANES

NC = 2
NS = 16
NW = NC * NS
B_PER_W = BATCH // NW

CHUNK = 50
NCH = SEQ // CHUNK
CPW = B_PER_W * NCH


def _sc_pool_body(idx_hbm, table_hbm, out_hbm, idx_v, buf0, buf1, stage_v,
                  sem0, sem1):
    wid = lax.axis_index("s") * NC + lax.axis_index("c")
    row0 = wid * B_PER_W
    pltpu.sync_copy(idx_hbm.at[pl.ds(wid * CPW, CPW)], idx_v)

    zero = jnp.zeros((LANES,), jnp.float32)

    def zero_buf(buf):
        def zb(k, c):
            for g in range(NREG):
                buf[k, pl.ds(g * LANES, LANES)] = zero
            return c
        lax.fori_loop(0, CHUNK, zb, 0)

    def fire(r, buf, sem):
        for p in range(NCH):
            pltpu.async_copy(
                table_hbm.at[idx_v.at[r * NCH + p]], buf, sem, add=True)

    def process(r, buf, sem):
        for p in range(NCH):
            pltpu.make_async_copy(table_hbm.at[idx_v.at[0]], buf, sem).wait()

        def acc_body(k, accs):
            new = []
            for g in range(NREG):
                sl = pl.ds(g * LANES, LANES)
                new.append(accs[g] + buf[k, sl])
                buf[k, sl] = zero
            return tuple(new)

        accs = lax.fori_loop(0, CHUNK, acc_body,
                             tuple(zero for _ in range(NREG)))
        for g in range(NREG):
            stage_v[r, pl.ds(g * LANES, LANES)] = accs[g]

    zero_buf(buf0)
    zero_buf(buf1)
    fire(0, buf0, sem0)

    def pair_body(rp, c):
        a = 2 * rp
        fire(a + 1, buf1, sem1)
        process(a, buf0, sem0)
        fire(a + 2, buf0, sem0)
        process(a + 1, buf1, sem1)
        return c

    lax.fori_loop(0, B_PER_W // 2 - 1, pair_body, 0)
    fire(B_PER_W - 1, buf1, sem1)
    process(B_PER_W - 2, buf0, sem0)
    process(B_PER_W - 1, buf1, sem1)

    pltpu.sync_copy(stage_v, out_hbm.at[pl.ds(row0, B_PER_W)])


@functools.lru_cache(maxsize=1)
def _make_sc_pool():
    return pl.kernel(
        _sc_pool_body,
        out_type=jax.ShapeDtypeStruct((BATCH, EMBED_DIM), jnp.float32),
        mesh=plsc.VectorSubcoreMesh(core_axis_name="c", subcore_axis_name="s"),
        scratch_types=[
            pltpu.VMEM((CPW, CHUNK), jnp.int32),
            pltpu.VMEM((CHUNK, EMBED_DIM), jnp.float32),
            pltpu.VMEM((CHUNK, EMBED_DIM), jnp.float32),
            pltpu.VMEM((B_PER_W, EMBED_DIM), jnp.float32),
            pltpu.SemaphoreType.DMA,
            pltpu.SemaphoreType.DMA,
        ],
    )


def _tc_head_body(x_ref, w_ref, b_ref, o_ref):
    x = x_ref[...]
    w = w_ref[...]
    logits = lax.dot_general(
        x, w, (((1,), (1,)), ((), ())), preferred_element_type=jnp.float32
    )
    logits = logits + b_ref[...]
    m = jnp.max(logits, axis=1, keepdims=True)
    s = logits - m
    lse = jnp.log(jnp.sum(jnp.exp(s), axis=1, keepdims=True))
    o_ref[...] = s - lse


def _tc_head(pooled, W, b2d):
    n_classes = W.shape[0]
    bm = 512
    grid = (BATCH // bm,)
    return pl.pallas_call(
        _tc_head_body,
        grid=grid,
        in_specs=[
            pl.BlockSpec((bm, EMBED_DIM), lambda i: (i, 0)),
            pl.BlockSpec((n_classes, EMBED_DIM), lambda i: (0, 0)),
            pl.BlockSpec((1, n_classes), lambda i: (0, 0)),
        ],
        out_specs=pl.BlockSpec((bm, n_classes), lambda i: (i, 0)),
        out_shape=jax.ShapeDtypeStruct((BATCH, n_classes), jnp.float32),
    )(pooled, W, b2d)


def kernel(inputs, emb_table, W, b):
    idx2d = inputs.reshape(BATCH * SEQ // CHUNK, CHUNK).astype(jnp.int32)
    pooled = _make_sc_pool()(idx2d, emb_table)
    return _tc_head(pooled, W, b.reshape(1, -1))


# speedup vs baseline: 13.5482x; 1.1542x over previous
import functools

import jax
import jax.numpy as jnp
from jax import lax
from jax.experimental import pallas as pl
from jax.experimental.pallas import tpu as pltpu
from jax.experimental.pallas import tpu_sc as plsc

BATCH = 4096
SEQ = 200
EMBED_DIM = 128
LANES = 16
NREG = EMBED_DIM // LANES

NC = 2
NS = 16
NW = NC * NS
B_PER_W = BATCH // NW

CHUNK = 50
NCH = SEQ // CHUNK
CPW = B_PER_W * NCH


NBUF = 4


def _sc_pool_body(idx_hbm, table_hbm, out_hbm, idx_v, buf0, buf1, buf2, buf3,
                  stage_v, sem0, sem1, sem2, sem3):
    bufs = (buf0, buf1, buf2, buf3)
    sems = (sem0, sem1, sem2, sem3)
    wid = lax.axis_index("s") * NC + lax.axis_index("c")
    row0 = wid * B_PER_W
    pltpu.sync_copy(idx_hbm.at[pl.ds(wid * CPW, CPW)], idx_v)

    zero = jnp.zeros((LANES,), jnp.float32)

    def zero_buf(buf):
        def zb(k, c):
            for g in range(NREG):
                buf[k, pl.ds(g * LANES, LANES)] = zero
            return c
        lax.fori_loop(0, CHUNK, zb, 0)

    def fire(r, buf, sem):
        for p in range(NCH):
            pltpu.async_copy(
                table_hbm.at[idx_v.at[r * NCH + p]], buf, sem, add=True)

    def process(r, buf, sem):
        for p in range(NCH):
            pltpu.make_async_copy(table_hbm.at[idx_v.at[0]], buf, sem).wait()

        def acc_body(k, accs):
            new = []
            for g in range(NREG):
                sl = pl.ds(g * LANES, LANES)
                new.append(accs[g] + buf[k, sl])
                buf[k, sl] = zero
            return tuple(new)

        accs = lax.fori_loop(0, CHUNK, acc_body,
                             tuple(zero for _ in range(NREG)))
        for g in range(NREG):
            stage_v[r, pl.ds(g * LANES, LANES)] = accs[g]

    for i in range(NBUF):
        zero_buf(bufs[i])
    for i in range(NBUF):
        fire(i, bufs[i], sems[i])

    def quad_body(rq, c):
        r = NBUF * rq
        for i in range(NBUF):
            process(r + i, bufs[i], sems[i])
            fire(r + i + NBUF, bufs[i], sems[i])
        return c

    lax.fori_loop(0, B_PER_W // NBUF - 1, quad_body, 0)
    r_last = B_PER_W - NBUF
    for i in range(NBUF):
        process(r_last + i, bufs[i], sems[i])

    pltpu.sync_copy(stage_v, out_hbm.at[pl.ds(row0, B_PER_W)])


@functools.lru_cache(maxsize=1)
def _make_sc_pool():
    return pl.kernel(
        _sc_pool_body,
        out_type=jax.ShapeDtypeStruct((BATCH, EMBED_DIM), jnp.float32),
        mesh=plsc.VectorSubcoreMesh(core_axis_name="c", subcore_axis_name="s"),
        scratch_types=(
            [pltpu.VMEM((CPW, CHUNK), jnp.int32)]
            + [pltpu.VMEM((CHUNK, EMBED_DIM), jnp.float32)
               for _ in range(NBUF)]
            + [pltpu.VMEM((B_PER_W, EMBED_DIM), jnp.float32)]
            + [pltpu.SemaphoreType.DMA for _ in range(NBUF)]
        ),
    )


def _tc_head_body(x_ref, w_ref, b_ref, o_ref):
    x = x_ref[...]
    w = w_ref[...]
    logits = lax.dot_general(
        x, w, (((1,), (1,)), ((), ())), preferred_element_type=jnp.float32
    )
    logits = logits + b_ref[...]
    m = jnp.max(logits, axis=1, keepdims=True)
    s = logits - m
    lse = jnp.log(jnp.sum(jnp.exp(s), axis=1, keepdims=True))
    o_ref[...] = s - lse


def _tc_head(pooled, W, b2d):
    n_classes = W.shape[0]
    bm = 512
    grid = (BATCH // bm,)
    return pl.pallas_call(
        _tc_head_body,
        grid=grid,
        in_specs=[
            pl.BlockSpec((bm, EMBED_DIM), lambda i: (i, 0)),
            pl.BlockSpec((n_classes, EMBED_DIM), lambda i: (0, 0)),
            pl.BlockSpec((1, n_classes), lambda i: (0, 0)),
        ],
        out_specs=pl.BlockSpec((bm, n_classes), lambda i: (i, 0)),
        out_shape=jax.ShapeDtypeStruct((BATCH, n_classes), jnp.float32),
    )(pooled, W, b2d)


def kernel(inputs, emb_table, W, b):
    idx2d = inputs.reshape(BATCH * SEQ // CHUNK, CHUNK).astype(jnp.int32)
    pooled = _make_sc_pool()(idx2d, emb_table)
    return _tc_head(pooled, W, b.reshape(1, -1))


# speedup vs baseline: 13.5967x; 1.0036x over previous
import functools

import jax
import jax.numpy as jnp
from jax import lax
from jax.experimental import pallas as pl
from jax.experimental.pallas import tpu as pltpu
from jax.experimental.pallas import tpu_sc as plsc

BATCH = 4096
SEQ = 200
EMBED_DIM = 128
LANES = 16
NREG = EMBED_DIM // LANES

NC = 2
NS = 16
NW = NC * NS
B_PER_W = BATCH // NW

CHUNK = 40
NCH = SEQ // CHUNK
IPW = B_PER_W * SEQ


NBUF = 4


def _sc_pool_body(idx_hbm, table_hbm, out_hbm, idx_v, buf0, buf1, buf2, buf3,
                  stage_v, sem0, sem1, sem2, sem3):
    bufs = (buf0, buf1, buf2, buf3)
    sems = (sem0, sem1, sem2, sem3)
    wid = lax.axis_index("s") * NC + lax.axis_index("c")
    row0 = wid * B_PER_W
    pltpu.sync_copy(idx_hbm.at[pl.ds(wid * IPW, IPW)], idx_v)

    zero = jnp.zeros((LANES,), jnp.float32)

    def zero_buf(buf):
        def zb(k, c):
            for g in range(NREG):
                buf[k, pl.ds(g * LANES, LANES)] = zero
            return c
        lax.fori_loop(0, CHUNK, zb, 0)

    def fire(r, buf, sem):
        for p in range(NCH):
            pltpu.async_copy(
                table_hbm.at[idx_v.at[pl.ds(r * SEQ + p * CHUNK, CHUNK)]],
                buf, sem, add=True)

    def process(r, buf, sem):
        for p in range(NCH):
            pltpu.make_async_copy(
                table_hbm.at[idx_v.at[pl.ds(0, CHUNK)]], buf, sem).wait()

        def acc_body(k, accs):
            new = []
            for g in range(NREG):
                sl = pl.ds(g * LANES, LANES)
                new.append(accs[g] + buf[k, sl])
                buf[k, sl] = zero
            return tuple(new)

        accs = lax.fori_loop(0, CHUNK, acc_body,
                             tuple(zero for _ in range(NREG)))
        for g in range(NREG):
            stage_v[r, pl.ds(g * LANES, LANES)] = accs[g]

    for i in range(NBUF):
        zero_buf(bufs[i])
    for i in range(NBUF):
        fire(i, bufs[i], sems[i])

    def quad_body(rq, c):
        r = NBUF * rq
        for i in range(NBUF):
            process(r + i, bufs[i], sems[i])
            fire(r + i + NBUF, bufs[i], sems[i])
        return c

    lax.fori_loop(0, B_PER_W // NBUF - 1, quad_body, 0)
    r_last = B_PER_W - NBUF
    for i in range(NBUF):
        process(r_last + i, bufs[i], sems[i])

    pltpu.sync_copy(stage_v, out_hbm.at[pl.ds(row0, B_PER_W)])


@functools.lru_cache(maxsize=1)
def _make_sc_pool():
    return pl.kernel(
        _sc_pool_body,
        out_type=jax.ShapeDtypeStruct((BATCH, EMBED_DIM), jnp.float32),
        mesh=plsc.VectorSubcoreMesh(core_axis_name="c", subcore_axis_name="s"),
        scratch_types=(
            [pltpu.VMEM((IPW,), jnp.int32)]
            + [pltpu.VMEM((CHUNK, EMBED_DIM), jnp.float32)
               for _ in range(NBUF)]
            + [pltpu.VMEM((B_PER_W, EMBED_DIM), jnp.float32)]
            + [pltpu.SemaphoreType.DMA for _ in range(NBUF)]
        ),
    )


def _tc_head_body(x_ref, w_ref, b_ref, o_ref):
    x = x_ref[...]
    w = w_ref[...]
    logits = lax.dot_general(
        x, w, (((1,), (1,)), ((), ())), preferred_element_type=jnp.float32
    )
    logits = logits + b_ref[...]
    m = jnp.max(logits, axis=1, keepdims=True)
    s = logits - m
    lse = jnp.log(jnp.sum(jnp.exp(s), axis=1, keepdims=True))
    o_ref[...] = s - lse


def _tc_head(pooled, W, b2d):
    n_classes = W.shape[0]
    bm = 512
    grid = (BATCH // bm,)
    return pl.pallas_call(
        _tc_head_body,
        grid=grid,
        in_specs=[
            pl.BlockSpec((bm, EMBED_DIM), lambda i: (i, 0)),
            pl.BlockSpec((n_classes, EMBED_DIM), lambda i: (0, 0)),
            pl.BlockSpec((1, n_classes), lambda i: (0, 0)),
        ],
        out_specs=pl.BlockSpec((bm, n_classes), lambda i: (i, 0)),
        out_shape=jax.ShapeDtypeStruct((BATCH, n_classes), jnp.float32),
    )(pooled, W, b2d)


def kernel(inputs, emb_table, W, b):
    idx_flat = inputs.reshape(BATCH * SEQ).astype(jnp.int32)
    pooled = _make_sc_pool()(idx_flat, emb_table)
    return _tc_head(pooled, W, b.reshape(1, -1))


# speedup vs baseline: 13.9818x; 1.0283x over previous
import functools

import jax
import jax.numpy as jnp
from jax import lax
from jax.experimental import pallas as pl
from jax.experimental.pallas import tpu as pltpu
from jax.experimental.pallas import tpu_sc as plsc

BATCH = 4096
SEQ = 200
EMBED_DIM = 128
LANES = 16
NREG = EMBED_DIM // LANES

NC = 2
NS = 16
NW = NC * NS
B_PER_W = BATCH // NW

CHUNK = 40
NCH = SEQ // CHUNK
IPW = B_PER_W * SEQ


NBUF = 8


def _sc_pool_body(idx_hbm, table_hbm, out_hbm, idx_v, buf0, buf1, buf2, buf3,
                  buf4, buf5, buf6, buf7, stage_v, sem0, sem1, sem2, sem3,
                  sem4, sem5, sem6, sem7):
    bufs = (buf0, buf1, buf2, buf3, buf4, buf5, buf6, buf7)
    sems = (sem0, sem1, sem2, sem3, sem4, sem5, sem6, sem7)
    wid = lax.axis_index("s") * NC + lax.axis_index("c")
    row0 = wid * B_PER_W
    pltpu.sync_copy(idx_hbm.at[pl.ds(wid * IPW, IPW)], idx_v)

    zero = jnp.zeros((LANES,), jnp.float32)

    def zero_buf(buf):
        def zb(k, c):
            for g in range(NREG):
                buf[k, pl.ds(g * LANES, LANES)] = zero
            return c
        lax.fori_loop(0, CHUNK, zb, 0)

    def fire(r, buf, sem):
        for p in range(NCH):
            pltpu.async_copy(
                table_hbm.at[idx_v.at[pl.ds(r * SEQ + p * CHUNK, CHUNK)]],
                buf, sem, add=True)

    def process(r, buf, sem):
        for p in range(NCH):
            pltpu.make_async_copy(
                table_hbm.at[idx_v.at[pl.ds(0, CHUNK)]], buf, sem).wait()

        def acc_body(k, accs):
            new = []
            for g in range(NREG):
                sl = pl.ds(g * LANES, LANES)
                new.append(accs[g] + buf[k, sl])
                buf[k, sl] = zero
            return tuple(new)

        accs = lax.fori_loop(0, CHUNK, acc_body,
                             tuple(zero for _ in range(NREG)))
        for g in range(NREG):
            stage_v[r, pl.ds(g * LANES, LANES)] = accs[g]

    for i in range(NBUF):
        zero_buf(bufs[i])
    for i in range(NBUF):
        fire(i, bufs[i], sems[i])

    def quad_body(rq, c):
        r = NBUF * rq
        for i in range(NBUF):
            process(r + i, bufs[i], sems[i])
            fire(r + i + NBUF, bufs[i], sems[i])
        return c

    lax.fori_loop(0, B_PER_W // NBUF - 1, quad_body, 0)
    r_last = B_PER_W - NBUF
    for i in range(NBUF):
        process(r_last + i, bufs[i], sems[i])

    pltpu.sync_copy(stage_v, out_hbm.at[pl.ds(row0, B_PER_W)])


@functools.lru_cache(maxsize=1)
def _make_sc_pool():
    return pl.kernel(
        _sc_pool_body,
        out_type=jax.ShapeDtypeStruct((BATCH, EMBED_DIM), jnp.float32),
        mesh=plsc.VectorSubcoreMesh(core_axis_name="c", subcore_axis_name="s"),
        scratch_types=(
            [pltpu.VMEM((IPW,), jnp.int32)]
            + [pltpu.VMEM((CHUNK, EMBED_DIM), jnp.float32)
               for _ in range(NBUF)]
            + [pltpu.VMEM((B_PER_W, EMBED_DIM), jnp.float32)]
            + [pltpu.SemaphoreType.DMA for _ in range(NBUF)]
        ),
    )


def _tc_head_body(x_ref, w_ref, b_ref, o_ref):
    x = x_ref[...]
    w = w_ref[...]
    logits = lax.dot_general(
        x, w, (((1,), (1,)), ((), ())), preferred_element_type=jnp.float32
    )
    logits = logits + b_ref[...]
    m = jnp.max(logits, axis=1, keepdims=True)
    s = logits - m
    lse = jnp.log(jnp.sum(jnp.exp(s), axis=1, keepdims=True))
    o_ref[...] = s - lse


def _tc_head(pooled, W, b2d):
    n_classes = W.shape[0]
    bm = 512
    grid = (BATCH // bm,)
    return pl.pallas_call(
        _tc_head_body,
        grid=grid,
        in_specs=[
            pl.BlockSpec((bm, EMBED_DIM), lambda i: (i, 0)),
            pl.BlockSpec((n_classes, EMBED_DIM), lambda i: (0, 0)),
            pl.BlockSpec((1, n_classes), lambda i: (0, 0)),
        ],
        out_specs=pl.BlockSpec((bm, n_classes), lambda i: (i, 0)),
        out_shape=jax.ShapeDtypeStruct((BATCH, n_classes), jnp.float32),
    )(pooled, W, b2d)


def kernel(inputs, emb_table, W, b):
    idx_flat = inputs.reshape(BATCH * SEQ).astype(jnp.int32)
    pooled = _make_sc_pool()(idx_flat, emb_table)
    return _tc_head(pooled, W, b.reshape(1, -1))


# speedup vs baseline: 15.7212x; 1.1244x over previous
import functools

import jax
import jax.numpy as jnp
from jax import lax
from jax.experimental import pallas as pl
from jax.experimental.pallas import tpu as pltpu
from jax.experimental.pallas import tpu_sc as plsc

BATCH = 4096
SEQ = 200
EMBED_DIM = 128
LANES = 16
NREG = EMBED_DIM // LANES

NC = 2
NS = 16
NW = NC * NS

NSPLIT = 2
HALF = BATCH // NSPLIT
B_PER_W = HALF // NW

N_CLASSES = 1000
BM = 512


@functools.lru_cache(maxsize=NSPLIT)
def _make_sc_pool(split):
    def body(idxt_hbm, table_hbm, out_hbm, idx_v, buf, sem):
        wid = lax.axis_index("s") * NC + lax.axis_index("c")
        blk0 = split * HALF + (wid // 2) * 128
        pltpu.sync_copy(
            idxt_hbm.at[pl.ds(0, SEQ), pl.ds(blk0, 128)], idx_v)

        zero = jnp.zeros((LANES,), jnp.float32)

        def zb(k, c):
            for g in range(NREG):
                buf[k, pl.ds(g * LANES, LANES)] = zero
            return c

        lax.fori_loop(0, B_PER_W, zb, 0)

        def fire_all(off):
            def fire_s(s, c):
                pltpu.async_copy(
                    table_hbm.at[idx_v.at[s, pl.ds(off, B_PER_W)]],
                    buf, sem, add=True)
                return c
            lax.fori_loop(0, SEQ, fire_s, 0)

        @pl.when(wid % 2 == 0)
        def _():
            fire_all(0)

        @pl.when(wid % 2 == 1)
        def _():
            fire_all(B_PER_W)

        def drain_s(s, c):
            pltpu.make_async_copy(
                table_hbm.at[idx_v.at[0, pl.ds(0, B_PER_W)]],
                buf, sem).wait()
            return c

        lax.fori_loop(0, SEQ, drain_s, 0)
        pltpu.sync_copy(buf, out_hbm.at[pl.ds(wid * B_PER_W, B_PER_W)])

    return pl.kernel(
        body,
        out_type=jax.ShapeDtypeStruct((HALF, EMBED_DIM), jnp.float32),
        mesh=plsc.VectorSubcoreMesh(core_axis_name="c", subcore_axis_name="s"),
        compiler_params=pltpu.CompilerParams(use_tc_tiling_on_sc=True),
        scratch_types=[
            pltpu.VMEM((SEQ, 128), jnp.int32),
            pltpu.VMEM((B_PER_W, EMBED_DIM), jnp.float32),
            pltpu.SemaphoreType.DMA,
        ],
    )


def _tc_head_body(x_ref, w_ref, b_ref, *rest):
    o_ref = rest[-1]
    x = x_ref[...]
    w = w_ref[...]
    logits = lax.dot_general(
        w, x, (((1,), (1,)), ((), ())), preferred_element_type=jnp.float32
    )
    logits = logits + b_ref[...]
    m = jnp.max(logits, axis=0, keepdims=True)
    s = logits - m
    lse = jnp.log(jnp.sum(jnp.exp(s), axis=0, keepdims=True))
    o_ref[...] = s - lse


def _tc_head(pooled_half, W, bcol, prev, split):
    grid = (HALF // BM,)
    off = split * (HALF // BM)
    in_specs = [
        pl.BlockSpec((BM, EMBED_DIM), lambda i: (i, 0)),
        pl.BlockSpec((N_CLASSES, EMBED_DIM), lambda i: (0, 0)),
        pl.BlockSpec((N_CLASSES, 1), lambda i: (0, 0)),
    ]
    operands = [pooled_half, W, bcol]
    kwargs = {}
    if prev is not None:
        in_specs.append(pl.BlockSpec(memory_space=pl.ANY))
        operands.append(prev)
        kwargs["input_output_aliases"] = {3: 0}
    return pl.pallas_call(
        _tc_head_body,
        grid=grid,
        in_specs=in_specs,
        out_specs=pl.BlockSpec((N_CLASSES, BM), lambda i, off=off: (0, i + off)),
        out_shape=jax.ShapeDtypeStruct((N_CLASSES, BATCH), jnp.float32),
        **kwargs,
    )(*operands)


def kernel(inputs, emb_table, W, b):
    idx_t = inputs.T
    bcol = b.reshape(-1, 1)
    out_t = None
    for split in range(NSPLIT):
        pooled = _make_sc_pool(split)(idx_t, emb_table)
        out_t = _tc_head(pooled, W, bcol, out_t, split)
    return out_t.T


# speedup vs baseline: 15.9440x; 1.0142x over previous
import functools

import jax
import jax.numpy as jnp
from jax import lax
from jax.experimental import pallas as pl
from jax.experimental.pallas import tpu as pltpu
from jax.experimental.pallas import tpu_sc as plsc

BATCH = 4096
SEQ = 200
EMBED_DIM = 128
LANES = 16
NREG = EMBED_DIM // LANES

NC = 2
NS = 16
NW = NC * NS
B_PER_W = BATCH // NW


def _sc_pool_body(idxt_hbm, table_hbm, out_hbm, idx_v, buf, sem, stage_sem):
    wid = lax.axis_index("s") * NC + lax.axis_index("c")
    n0 = wid * B_PER_W
    stage = pltpu.async_copy(
        idxt_hbm.at[pl.ds(0, SEQ), pl.ds(n0, B_PER_W)], idx_v, stage_sem)

    zero = jnp.zeros((LANES,), jnp.float32)

    def zb(k, c):
        for g in range(NREG):
            buf[k, pl.ds(g * LANES, LANES)] = zero
        return c

    lax.fori_loop(0, B_PER_W, zb, 0)
    stage.wait()

    def fire_s(s, c):
        pltpu.async_copy(table_hbm.at[idx_v.at[s]], buf, sem, add=True)
        return c

    lax.fori_loop(0, SEQ, fire_s, 0)

    def drain_s(s, c):
        pltpu.make_async_copy(table_hbm.at[idx_v.at[0]], buf, sem).wait()
        return c

    lax.fori_loop(0, SEQ, drain_s, 0)
    pltpu.sync_copy(buf, out_hbm.at[pl.ds(n0, B_PER_W)])


@functools.lru_cache(maxsize=1)
def _make_sc_pool():
    return pl.kernel(
        _sc_pool_body,
        out_type=jax.ShapeDtypeStruct((BATCH, EMBED_DIM), jnp.float32),
        mesh=plsc.VectorSubcoreMesh(core_axis_name="c", subcore_axis_name="s"),
        compiler_params=pltpu.CompilerParams(use_tc_tiling_on_sc=True),
        scratch_types=[
            pltpu.VMEM((SEQ, B_PER_W), jnp.int32),
            pltpu.VMEM((B_PER_W, EMBED_DIM), jnp.float32),
            pltpu.SemaphoreType.DMA,
            pltpu.SemaphoreType.DMA,
        ],
    )


def _tc_head_body(x_ref, w_ref, b_ref, o_ref):
    x = x_ref[...]
    w = w_ref[...]
    logits = lax.dot_general(
        w, x, (((1,), (1,)), ((), ())), preferred_element_type=jnp.float32
    )
    logits = logits + b_ref[...]
    m = jnp.max(logits, axis=0, keepdims=True)
    s = logits - m
    lse = jnp.log(jnp.sum(jnp.exp(s), axis=0, keepdims=True))
    o_ref[...] = s - lse


def _tc_head(pooled, W, bcol):
    n_classes = W.shape[0]
    bm = 512
    grid = (BATCH // bm,)
    out_t = pl.pallas_call(
        _tc_head_body,
        grid=grid,
        in_specs=[
            pl.BlockSpec((bm, EMBED_DIM), lambda i: (i, 0)),
            pl.BlockSpec((n_classes, EMBED_DIM), lambda i: (0, 0)),
            pl.BlockSpec((n_classes, 1), lambda i: (0, 0)),
        ],
        out_specs=pl.BlockSpec((n_classes, bm), lambda i: (0, i)),
        out_shape=jax.ShapeDtypeStruct((n_classes, BATCH), jnp.float32),
    )(pooled, W, bcol)
    return out_t.T


def kernel(inputs, emb_table, W, b):
    pooled = _make_sc_pool()(inputs.T, emb_table)
    return _tc_head(pooled, W, b.reshape(-1, 1))


# speedup vs baseline: 16.0939x; 1.0094x over previous
import functools

import jax
import jax.numpy as jnp
from jax import lax
from jax.experimental import pallas as pl
from jax.experimental.pallas import tpu as pltpu
from jax.experimental.pallas import tpu_sc as plsc

BATCH = 4096
SEQ = 200
EMBED_DIM = 128
LANES = 16
NREG = EMBED_DIM // LANES

NC = 2
NS = 16
NW = NC * NS
B_PER_W = BATCH // NW


def _sc_pool_body(idxt_hbm, table_hbm, out_hbm, idx_v, buf, sem, stage_sem):
    wid = lax.axis_index("s") * NC + lax.axis_index("c")
    n0 = wid * B_PER_W
    stage = pltpu.async_copy(
        idxt_hbm.at[pl.ds(0, SEQ), pl.ds(n0, B_PER_W)], idx_v, stage_sem)

    zero = jnp.zeros((LANES,), jnp.float32)

    def zb(k, c):
        for g in range(NREG):
            buf[k, pl.ds(g * LANES, LANES)] = zero
        return c

    lax.fori_loop(0, B_PER_W, zb, 0)
    stage.wait()

    def fire_s(s, c):
        pltpu.async_copy(table_hbm.at[idx_v.at[s]], buf, sem, add=True)
        return c

    lax.fori_loop(0, SEQ, fire_s, 0)

    def drain_s(s, c):
        pltpu.make_async_copy(table_hbm.at[idx_v.at[0]], buf, sem).wait()
        return c

    lax.fori_loop(0, SEQ, drain_s, 0)
    pltpu.sync_copy(buf, out_hbm.at[pl.ds(n0, B_PER_W)])


@functools.lru_cache(maxsize=1)
def _make_sc_pool():
    return pl.kernel(
        _sc_pool_body,
        out_type=jax.ShapeDtypeStruct((BATCH, EMBED_DIM), jnp.float32),
        mesh=plsc.VectorSubcoreMesh(core_axis_name="c", subcore_axis_name="s"),
        compiler_params=pltpu.CompilerParams(use_tc_tiling_on_sc=True),
        scratch_types=[
            pltpu.VMEM((SEQ, B_PER_W), jnp.int32),
            pltpu.VMEM((B_PER_W, EMBED_DIM), jnp.float32),
            pltpu.SemaphoreType.DMA,
            pltpu.SemaphoreType.DMA,
        ],
    )


def _tc_head_body(x_ref, w_ref, b_ref, o_ref):
    x = x_ref[...]
    w = w_ref[...]
    logits = lax.dot_general(
        w, x, (((1,), (1,)), ((), ())), preferred_element_type=jnp.float32
    )
    logits = logits + b_ref[...]
    m = jnp.max(logits, axis=0, keepdims=True)
    s = logits - m
    lse = jnp.log(jnp.sum(jnp.exp(s), axis=0, keepdims=True))
    o_ref[...] = s - lse


def _tc_head(pooled, W, bcol):
    n_classes = W.shape[0]
    bm = 1024
    grid = (BATCH // bm,)
    out_t = pl.pallas_call(
        _tc_head_body,
        grid=grid,
        in_specs=[
            pl.BlockSpec((bm, EMBED_DIM), lambda i: (i, 0)),
            pl.BlockSpec((n_classes, EMBED_DIM), lambda i: (0, 0)),
            pl.BlockSpec((n_classes, 1), lambda i: (0, 0)),
        ],
        out_specs=pl.BlockSpec((n_classes, bm), lambda i: (0, i)),
        out_shape=jax.ShapeDtypeStruct((n_classes, BATCH), jnp.float32),
    )(pooled, W, bcol)
    return out_t.T


def kernel(inputs, emb_table, W, b):
    pooled = _make_sc_pool()(inputs.T, emb_table)
    return _tc_head(pooled, W, b.reshape(-1, 1))
